# baseline (speedup 1.0000x reference)
import functools

import jax
import jax.numpy as jnp
from jax import lax
from jax.experimental import pallas as pl
from jax.experimental.pallas import tpu as pltpu
from jax.experimental.pallas import tpu_sc as plsc

_F = 26
_V = 100000
_D = 16
_B = 4096
_H1 = 256
_H2 = 128

_NC = 2
_NS = 16
_NW = _NC * _NS
_BPW = _B // _NW
_LPW = _BPW * _F
_CHUNK = 128
_NCHUNK = _LPW // _CHUNK


def _gather_body(idx_hbm, table_hbm, out_hbm, idx_v, rows_v, sem):
    wid = lax.axis_index("s") * _NC + lax.axis_index("c")
    pltpu.sync_copy(idx_hbm.at[wid], idx_v)
    copies = []
    for j in range(_NCHUNK):
        copies.append(
            pltpu.async_copy(
                table_hbm.at[idx_v.at[j]],
                rows_v.at[pl.ds(j * _CHUNK, _CHUNK)],
                sem,
            )
        )
    for c in copies:
        c.wait()
    pltpu.sync_copy(rows_v, out_hbm.at[pl.ds(wid * _LPW, _LPW)])


@functools.lru_cache(maxsize=1)
def _make_sc_gather():
    return functools.partial(
        pl.kernel,
        out_type=jax.ShapeDtypeStruct((_B * _F, _D), jnp.float32),
        mesh=plsc.VectorSubcoreMesh(core_axis_name="c", subcore_axis_name="s"),
        scratch_types=[
            pltpu.VMEM((_NCHUNK, _CHUNK), jnp.int32),
            pltpu.VMEM((_LPW, _D), jnp.float32),
            pltpu.SemaphoreType.DMA,
        ],
        compiler_params=pltpu.CompilerParams(use_tc_tiling_on_sc=False),
    )(_gather_body)


def _mlp_body(x_ref, bn0_g, bn0_b, w1, b1, bn1_g, bn1_b, w2, b2,
              bn2_g, bn2_b, w3, b3, out_ref):
    eps = 1e-5
    x = x_ref[...]
    mu = jnp.mean(x, axis=0, keepdims=True)
    var = jnp.mean((x - mu) * (x - mu), axis=0, keepdims=True)
    x = (x - mu) * lax.rsqrt(var + eps) * bn0_g[...] + bn0_b[...]
    h = jnp.maximum(
        jnp.dot(x, w1[...], preferred_element_type=jnp.float32) + b1[...], 0.0)
    mu = jnp.mean(h, axis=0, keepdims=True)
    var = jnp.mean((h - mu) * (h - mu), axis=0, keepdims=True)
    h = (h - mu) * lax.rsqrt(var + eps) * bn1_g[...] + bn1_b[...]
    h = jnp.maximum(
        jnp.dot(h, w2[...], preferred_element_type=jnp.float32) + b2[...], 0.0)
    mu = jnp.mean(h, axis=0, keepdims=True)
    var = jnp.mean((h - mu) * (h - mu), axis=0, keepdims=True)
    h = (h - mu) * lax.rsqrt(var + eps) * bn2_g[...] + bn2_b[...]
    logit = jnp.sum(h * w3[...], axis=1, keepdims=True) + b3[0, 0]
    out_ref[...] = 1.0 / (1.0 + jnp.exp(-logit))


def _run_mlp(x, bn0_g, bn0_b, w1, b1, bn1_g, bn1_b, w2, b2,
             bn2_g, bn2_b, w3, b3, *, interpret=False):
    return pl.pallas_call(
        _mlp_body,
        out_shape=jax.ShapeDtypeStruct((_B, 1), jnp.float32),
        interpret=interpret,
    )(x, bn0_g, bn0_b, w1, b1, bn1_g, bn1_b, w2, b2, bn2_g, bn2_b, w3, b3)


def kernel(x_cat, emb_tables, bn0_g, bn0_b, W1, b1, bn1_g, bn1_b,
           W2, b2, bn2_g, bn2_b, W3, b3):
    flat_idx = (x_cat.astype(jnp.int32)
                + (jnp.arange(_F, dtype=jnp.int32) * _V)[None, :])
    flat_idx = flat_idx.reshape(_NW, _NCHUNK, _CHUNK)
    table = emb_tables.reshape(_F * _V, _D)
    rows = _make_sc_gather()(flat_idx, table)
    x = rows.reshape(_B, _F * _D)
    out = _run_mlp(
        x,
        bn0_g.reshape(1, -1), bn0_b.reshape(1, -1),
        W1, b1.reshape(1, -1),
        bn1_g.reshape(1, -1), bn1_b.reshape(1, -1),
        W2, b2.reshape(1, -1),
        bn2_g.reshape(1, -1), bn2_b.reshape(1, -1),
        W3.reshape(1, _H2), b3.reshape(1, 1),
    )
    return out.reshape(_B)


# speedup vs baseline: 2.0846x; 2.0846x over previous
import functools

import jax
import jax.numpy as jnp
from jax import lax
from jax.experimental import pallas as pl
from jax.experimental.pallas import tpu as pltpu
from jax.experimental.pallas import tpu_sc as plsc

_F = 26
_V = 100000
_D = 16
_B = 4096
_H1 = 256
_H2 = 128

_NC = 2
_NS = 16
_NW = _NC * _NS
_BPW = _B // _NW
_LPW = _BPW * _F
_CHUNK = 128
_NCHUNK = _LPW // _CHUNK


_CH = 16
_NCH = _LPW // _CH

_DIV26_M = 20165
_DIV26_S = 19


def _gather_body(idx_hbm, table_hbm, out_hbm, iall, tiles, rows2, sem):
    wid = lax.axis_index("s") * _NC + lax.axis_index("c")
    pltpu.sync_copy(idx_hbm.at[wid], iall)
    iota = lax.iota(jnp.int32, 16)

    def step(t, carry):
        handles = []
        vecs = []
        for k in range(2):
            p0 = t * 32 + k * 16
            row = lax.shift_right_logical(p0, 7)
            colb = lax.rem(p0, _CHUNK)
            vec = iall[row, pl.ds(colb, 16)]
            vecs.append(vec)
            for m in range(16):
                q = lax.shift_right_logical(vec[m], 3)
                handles.append(pltpu.async_copy(
                    table_hbm.at[q], tiles.at[k * 16 + m], sem))
        for h in handles:
            h.wait()
        for k in range(2):
            p_vec = iota + (t * 32 + k * 16)
            s_vec = lax.bitwise_and(vecs[k], 7)
            g_vec = iota + (k * 16)
            brow = lax.shift_right_logical(p_vec * _DIV26_M, _DIV26_S)
            col0 = (p_vec - brow * _F) * _D
            for j in range(_D):
                vals = plsc.load_gather(
                    tiles, [g_vec, s_vec, jnp.full((16,), j, jnp.int32)])
                plsc.store_scatter(rows2, [brow, col0 + j], vals)
        return carry

    lax.fori_loop(0, _LPW // 32, step, 0, unroll=False)
    pltpu.sync_copy(rows2, out_hbm.at[pl.ds(wid * _BPW, _BPW)])


@functools.lru_cache(maxsize=1)
def _make_sc_gather():
    return functools.partial(
        pl.kernel,
        out_type=jax.ShapeDtypeStruct((_B, 512), jnp.float32),
        mesh=plsc.VectorSubcoreMesh(core_axis_name="c", subcore_axis_name="s"),
        scratch_types=[
            pltpu.VMEM((32, _CHUNK), jnp.int32),
            pltpu.VMEM((32, 8, _D), jnp.float32),
            pltpu.VMEM((_BPW, 512), jnp.float32),
            pltpu.SemaphoreType.DMA,
        ],
        compiler_params=pltpu.CompilerParams(needs_layout_passes=False),
    )(_gather_body)


def _mlp_body(x_ref, bn0_g, bn0_b, w1, b1, bn1_g, bn1_b, w2, b2,
              bn2_g, bn2_b, w3, b3, out_ref):
    eps = 1e-5
    x = x_ref[...][:, : _F * _D]
    mu = jnp.mean(x, axis=0, keepdims=True)
    var = jnp.mean((x - mu) * (x - mu), axis=0, keepdims=True)
    x = (x - mu) * lax.rsqrt(var + eps) * bn0_g[...] + bn0_b[...]
    h = jnp.maximum(
        jnp.dot(x, w1[...], preferred_element_type=jnp.float32) + b1[...], 0.0)
    mu = jnp.mean(h, axis=0, keepdims=True)
    var = jnp.mean((h - mu) * (h - mu), axis=0, keepdims=True)
    h = (h - mu) * lax.rsqrt(var + eps) * bn1_g[...] + bn1_b[...]
    h = jnp.maximum(
        jnp.dot(h, w2[...], preferred_element_type=jnp.float32) + b2[...], 0.0)
    mu = jnp.mean(h, axis=0, keepdims=True)
    var = jnp.mean((h - mu) * (h - mu), axis=0, keepdims=True)
    h = (h - mu) * lax.rsqrt(var + eps) * bn2_g[...] + bn2_b[...]
    logit = jnp.sum(h * w3[...], axis=1, keepdims=True) + b3[0, 0]
    out_ref[...] = 1.0 / (1.0 + jnp.exp(-logit))


def _run_mlp(x, bn0_g, bn0_b, w1, b1, bn1_g, bn1_b, w2, b2,
             bn2_g, bn2_b, w3, b3, *, interpret=False):
    return pl.pallas_call(
        _mlp_body,
        out_shape=jax.ShapeDtypeStruct((_B, 1), jnp.float32),
        interpret=interpret,
    )(x, bn0_g, bn0_b, w1, b1, bn1_g, bn1_b, w2, b2, bn2_g, bn2_b, w3, b3)


def kernel(x_cat, emb_tables, bn0_g, bn0_b, W1, b1, bn1_g, bn1_b,
           W2, b2, bn2_g, bn2_b, W3, b3):
    flat_idx = (x_cat.astype(jnp.int32)
                + (jnp.arange(_F, dtype=jnp.int32) * _V)[None, :])
    flat_idx = flat_idx.reshape(_NW, _NCHUNK, _CHUNK)
    flat_idx = jnp.pad(flat_idx, ((0, 0), (0, 32 - _NCHUNK), (0, 0)))
    table = emb_tables.reshape(_F * _V // 8, 8, _D)
    x = _make_sc_gather()(flat_idx, table)
    out = _run_mlp(
        x,
        bn0_g.reshape(1, -1), bn0_b.reshape(1, -1),
        W1, b1.reshape(1, -1),
        bn1_g.reshape(1, -1), bn1_b.reshape(1, -1),
        W2, b2.reshape(1, -1),
        bn2_g.reshape(1, -1), bn2_b.reshape(1, -1),
        W3.reshape(1, _H2), b3.reshape(1, 1),
    )
    return out.reshape(_B)
